# Initial kernel scaffold; baseline (speedup 1.0000x reference)
import jax
import jax.numpy as jnp
from jax.experimental import pallas as pl


def kernel(x, edge_index, batch, W_in, b_in, W_l, a_src, a_dst, b_l, ln_w, ln_b):
    raise NotImplementedError("write your pallas kernel here")


# speedup vs baseline: 30.1765x; 30.1765x over previous
import functools

import jax
import jax.numpy as jnp
from jax import lax
from jax.experimental import pallas as pl
from jax.experimental.pallas import tpu as pltpu
from jax.experimental.pallas import tpu_sc as plsc

N = 10000
E = 320000
D_IN = 128
HID = 256
H = 8
DH = 32
L = 3
G = 64

RB = 256
N_PAD = 10240
NBLK = N_PAD // RB
NC, NS = 2, 16
CB = 128
CHUNKS = 158
EP_T = CB * CHUNKS
E_PAD = EP_T * NS
HH = H // NC
FH = HID // NC
AW = FH + 16


def _in_body(x_ref, w_ref, b_ref, h_ref):
    h_ref[...] = (
        jnp.dot(x_ref[...], w_ref[...], preferred_element_type=jnp.float32)
        + b_ref[...]
    )


def _k_in(x, w, b):
    return pl.pallas_call(
        _in_body,
        grid=(NBLK,),
        in_specs=[
            pl.BlockSpec((RB, D_IN), lambda i: (i, 0)),
            pl.BlockSpec((D_IN, HID), lambda i: (0, 0)),
            pl.BlockSpec((1, HID), lambda i: (0, 0)),
        ],
        out_specs=pl.BlockSpec((RB, HID), lambda i: (i, 0)),
        out_shape=jax.ShapeDtypeStruct((N_PAD, HID), jnp.float32),
    )(x, w, b)


def _head_selector():
    kk = lax.broadcasted_iota(jnp.int32, (HID, H), 0) // DH
    hh = lax.broadcasted_iota(jnp.int32, (HID, H), 1)
    return (kk == hh).astype(jnp.float32)


def _pre_body(h_ref, w_ref, asv_ref, adv_ref, xpa_ref, adt_ref, cmax_ref,
              acc_ref):
    i = pl.program_id(0)
    h = h_ref[...]
    xp = jnp.dot(h, w_ref[...], preferred_element_type=jnp.float32)
    S = _head_selector()
    as8 = jnp.dot(xp * asv_ref[...], S, preferred_element_type=jnp.float32)
    ad8 = jnp.dot(xp * adv_ref[...], S, preferred_element_type=jnp.float32)

    z12 = jnp.zeros((RB, AW - FH - HH), jnp.float32)
    xpa0 = jnp.concatenate([xp[:, :FH], as8[:, :HH], z12], axis=1)
    xpa1 = jnp.concatenate([xp[:, FH:], as8[:, HH:], z12], axis=1)
    xpa_ref[...] = jnp.stack([xpa0, xpa1], axis=0)

    zad = jnp.zeros((RB, 16 - HH), jnp.float32)
    adt_ref[...] = jnp.stack(
        [jnp.concatenate([ad8[:, :HH], zad], axis=1),
         jnp.concatenate([ad8[:, HH:], zad], axis=1)], axis=0)

    @pl.when(i == 0)
    def _():
        acc_ref[...] = jnp.full((8, 16), -1e30, jnp.float32)

    acc_ref[0:1, 0:H] = jnp.maximum(acc_ref[0:1, 0:H],
                                    jnp.max(as8, axis=0, keepdims=True))
    acc_ref[1:2, 0:H] = jnp.maximum(acc_ref[1:2, 0:H],
                                    jnp.max(ad8, axis=0, keepdims=True))

    @pl.when(i == NBLK - 1)
    def _():
        cm = acc_ref[0:1, 0:H] + acc_ref[1:2, 0:H]
        cm = jnp.maximum(cm, 0.2 * cm)
        big = jnp.full((1, 16 - HH), 100.0, jnp.float32)
        row0 = jnp.concatenate([cm[:, :HH], big], axis=1)
        row1 = jnp.concatenate([cm[:, HH:], big], axis=1)
        cmax_ref[...] = jnp.concatenate([row0, row1], axis=0)


def _k_pre(h, w, asv, adv):
    return pl.pallas_call(
        _pre_body,
        grid=(NBLK,),
        in_specs=[
            pl.BlockSpec((RB, HID), lambda i: (i, 0)),
            pl.BlockSpec((HID, HID), lambda i: (0, 0)),
            pl.BlockSpec((1, HID), lambda i: (0, 0)),
            pl.BlockSpec((1, HID), lambda i: (0, 0)),
        ],
        out_specs=[
            pl.BlockSpec((NC, RB, AW), lambda i: (0, i, 0)),
            pl.BlockSpec((NC, RB, 16), lambda i: (0, i, 0)),
            pl.BlockSpec((NC, 16), lambda i: (0, 0)),
        ],
        out_shape=[
            jax.ShapeDtypeStruct((NC, N_PAD, AW), jnp.float32),
            jax.ShapeDtypeStruct((NC, N_PAD, 16), jnp.float32),
            jax.ShapeDtypeStruct((NC, 16), jnp.float32),
        ],
        scratch_shapes=[pltpu.VMEM((8, 16), jnp.float32)],
    )(h, w, asv, adv)


def _denom_expander():
    jj = lax.broadcasted_iota(jnp.int32, (HH, FH), 0)
    ff = lax.broadcasted_iota(jnp.int32, (HH, FH), 1) // DH
    return (jj == ff).astype(jnp.float32)


def _scaled_agg(agg):
    Sx = _denom_expander()
    outs = []
    for c in range(NC):
        num = agg[c, :, :FH]
        den = agg[c, :, FH:FH + HH] + 1e-16
        rec = jnp.dot(1.0 / den, Sx, preferred_element_type=jnp.float32)
        outs.append(num * rec)
    return jnp.concatenate(outs, axis=1)


def _post_common(h_ref, agg_ref, bl_ref, lnw_ref, lnb_ref):
    hmid = h_ref[...] + _scaled_agg(agg_ref[...]) + bl_ref[...]
    m = jnp.mean(hmid, axis=1, keepdims=True)
    d = hmid - m
    v = jnp.mean(d * d, axis=1, keepdims=True)
    hn = d * lax.rsqrt(v + 1e-5) * lnw_ref[...] + lnb_ref[...]
    return jnp.maximum(hn, 0.0)


def _post_body(h_ref, agg_ref, bl_ref, lnw_ref, lnb_ref, o_ref):
    o_ref[...] = _post_common(h_ref, agg_ref, bl_ref, lnw_ref, lnb_ref)


def _k_post(h, agg, bl, lnw, lnb):
    return pl.pallas_call(
        _post_body,
        grid=(NBLK,),
        in_specs=[
            pl.BlockSpec((RB, HID), lambda i: (i, 0)),
            pl.BlockSpec((NC, RB, AW), lambda i: (0, i, 0)),
            pl.BlockSpec((1, HID), lambda i: (0, 0)),
            pl.BlockSpec((1, HID), lambda i: (0, 0)),
            pl.BlockSpec((1, HID), lambda i: (0, 0)),
        ],
        out_specs=pl.BlockSpec((RB, HID), lambda i: (i, 0)),
        out_shape=jax.ShapeDtypeStruct((N_PAD, HID), jnp.float32),
    )(h, agg, bl, lnw, lnb)


def _pool_body(h_ref, agg_ref, bl_ref, lnw_ref, lnb_ref, batch_ref, o_ref,
               sum_ref, cnt_ref):
    i = pl.program_id(0)
    hn = _post_common(h_ref, agg_ref, bl_ref, lnw_ref, lnb_ref)

    bb = batch_ref[0]
    gg = lax.broadcasted_iota(jnp.int32, (G, RB), 0)
    P = (bb == gg).astype(jnp.float32)

    @pl.when(i == 0)
    def _():
        sum_ref[...] = jnp.zeros((G, HID), jnp.float32)
        cnt_ref[...] = jnp.zeros((G, 128), jnp.float32)

    sum_ref[...] += jnp.dot(P, hn, preferred_element_type=jnp.float32)
    cnt_ref[...] += jnp.dot(P, jnp.ones((RB, 128), jnp.float32),
                            preferred_element_type=jnp.float32)

    @pl.when(i == NBLK - 1)
    def _():
        c1 = jnp.maximum(cnt_ref[...], 1.0)
        o_ref[...] = sum_ref[...] / jnp.concatenate([c1, c1], axis=1)


def _k_pool(h, agg, bl, lnw, lnb, batch3):
    return pl.pallas_call(
        _pool_body,
        grid=(NBLK,),
        in_specs=[
            pl.BlockSpec((RB, HID), lambda i: (i, 0)),
            pl.BlockSpec((NC, RB, AW), lambda i: (0, i, 0)),
            pl.BlockSpec((1, HID), lambda i: (0, 0)),
            pl.BlockSpec((1, HID), lambda i: (0, 0)),
            pl.BlockSpec((1, HID), lambda i: (0, 0)),
            pl.BlockSpec((1, 1, RB), lambda i: (i, 0, 0)),
        ],
        out_specs=pl.BlockSpec((G, HID), lambda i: (0, 0)),
        out_shape=jax.ShapeDtypeStruct((G, HID), jnp.float32),
        scratch_shapes=[pltpu.VMEM((G, HID), jnp.float32),
                        pltpu.VMEM((G, 128), jnp.float32)],
    )(h, agg, bl, lnw, lnb, batch3)


_SC_MESH = plsc.VectorSubcoreMesh(
    core_axis_name="c", subcore_axis_name="s", num_cores=NC, num_subcores=NS)

_ROWS_PER = N_PAD // NS


@functools.partial(
    pl.kernel,
    out_type=jax.ShapeDtypeStruct((NC, N_PAD, AW), jnp.float32),
    mesh=_SC_MESH,
    compiler_params=pltpu.CompilerParams(use_tc_tiling_on_sc=False),
    scratch_types=[
        pltpu.VMEM_SHARED((N_PAD, AW), jnp.float32),
        pltpu.VMEM((CB, AW), jnp.float32),
        pltpu.VMEM((CB, 16), jnp.float32),
        pltpu.VMEM((CB,), jnp.int32),
        pltpu.VMEM((CB,), jnp.int32),
        pltpu.VMEM((16,), jnp.float32),
    ],
)
def _sc_edge_kernel(src_hbm, dst_hbm, xpa_hbm, adt_hbm, cmax_hbm, out_hbm,
                    agg_sp, sbuf, dbuf, sidx, didx, cbuf):
    c = lax.axis_index("c")
    s = lax.axis_index("s")

    @pl.loop(0, CB)
    def _zero(e):
        for k in range(AW // 16):
            sbuf[e, 16 * k:16 * (k + 1)] = jnp.zeros((16,), jnp.float32)

    for j in range(_ROWS_PER // CB):
        pltpu.sync_copy(sbuf, agg_sp.at[pl.ds(s * _ROWS_PER + j * CB, CB)])
    plsc.subcore_barrier()

    pltpu.sync_copy(cmax_hbm.at[c], cbuf)
    C = cbuf[...]

    base = s * EP_T

    @pl.loop(0, CHUNKS)
    def _chunk(i):
        off = base + i * CB
        pltpu.sync_copy(src_hbm.at[pl.ds(off, CB)], sidx)
        pltpu.sync_copy(dst_hbm.at[pl.ds(off, CB)], didx)
        pltpu.sync_copy(xpa_hbm.at[c].at[sidx], sbuf)
        pltpu.sync_copy(adt_hbm.at[c].at[didx], dbuf)

        @pl.loop(0, CB)
        def _edge(e):
            arow = sbuf[e, FH:FH + 16]
            drow = dbuf[e, :]
            sm = arow + drow
            sm = jnp.maximum(sm, 0.2 * sm)
            p16 = jnp.exp(sm - C)
            sbuf[e, FH:FH + 16] = p16
            for hh_ in range(HH):
                m = jnp.full((16,), p16[hh_], jnp.float32)
                for k2 in range(2):
                    k = hh_ * 2 + k2
                    sl = pl.ds(16 * k, 16)
                    sbuf[e, sl] = sbuf[e, sl] * m

        pltpu.sync_copy(sbuf, agg_sp.at[didx], add=True)

    plsc.subcore_barrier()
    for j in range(_ROWS_PER // CB):
        r0 = s * _ROWS_PER + j * CB
        pltpu.sync_copy(agg_sp.at[pl.ds(r0, CB)], out_hbm.at[c].at[pl.ds(r0, CB)])


def kernel(x, edge_index, batch, W_in, b_in, W_l, a_src, a_dst, b_l, ln_w,
           ln_b):
    f32 = jnp.float32
    x_pad = jnp.zeros((N_PAD, D_IN), f32).at[:N].set(x.astype(f32))
    pad_ids = jnp.full((E_PAD - E,), N, jnp.int32)
    srcp = jnp.concatenate([edge_index[0].astype(jnp.int32), pad_ids])
    dstp = jnp.concatenate([edge_index[1].astype(jnp.int32), pad_ids])
    batch3 = jnp.concatenate(
        [batch.astype(jnp.int32), jnp.full((N_PAD - N,), G, jnp.int32)]
    ).reshape(NBLK, 1, RB)

    b_in2 = b_in.reshape(1, HID).astype(f32)
    asv = a_src.reshape(L, HID).astype(f32)
    adv = a_dst.reshape(L, HID).astype(f32)

    h = _k_in(x_pad, W_in.astype(f32), b_in2)
    out = None
    for l in range(L):
        xpa, adt, cmax = _k_pre(h, W_l[l].astype(f32),
                                asv[l].reshape(1, HID),
                                adv[l].reshape(1, HID))
        agg = _sc_edge_kernel(srcp, dstp, xpa, adt, cmax)
        args = (h, agg, b_l[l].reshape(1, HID).astype(f32),
                ln_w[l].reshape(1, HID).astype(f32),
                ln_b[l].reshape(1, HID).astype(f32))
        if l < L - 1:
            h = _k_post(*args)
        else:
            out = _k_pool(*args, batch3)
    return out


# speedup vs baseline: 35.2423x; 1.1679x over previous
import functools

import jax
import jax.numpy as jnp
from jax import lax
from jax.experimental import pallas as pl
from jax.experimental.pallas import tpu as pltpu
from jax.experimental.pallas import tpu_sc as plsc

N = 10000
E = 320000
D_IN = 128
HID = 256
H = 8
DH = 32
L = 3
G = 64

RB = 128
N_PAD = 10112
NBLK = N_PAD // RB
NC, NS = 2, 16
CB = 128
CHUNKS = 160
GRP = 2
EP_T = CB * CHUNKS
E_PAD = EP_T * NS
HH = H // NC
FH = HID // NC
AW = FH + 16


def _in_body(x_ref, w_ref, b_ref, h_ref):
    h_ref[...] = (
        jnp.dot(x_ref[...], w_ref[...], preferred_element_type=jnp.float32)
        + b_ref[...]
    )


def _k_in(x, w, b):
    return pl.pallas_call(
        _in_body,
        grid=(NBLK,),
        in_specs=[
            pl.BlockSpec((RB, D_IN), lambda i: (i, 0)),
            pl.BlockSpec((D_IN, HID), lambda i: (0, 0)),
            pl.BlockSpec((1, HID), lambda i: (0, 0)),
        ],
        out_specs=pl.BlockSpec((RB, HID), lambda i: (i, 0)),
        out_shape=jax.ShapeDtypeStruct((N_PAD, HID), jnp.float32),
    )(x, w, b)


def _head_selector():
    kk = lax.broadcasted_iota(jnp.int32, (HID, H), 0) // DH
    hh = lax.broadcasted_iota(jnp.int32, (HID, H), 1)
    return (kk == hh).astype(jnp.float32)


def _pre_body(h_ref, w_ref, asv_ref, adv_ref, xpa_ref, adt_ref, cmax_ref,
              acc_ref):
    i = pl.program_id(0)
    h = h_ref[...]
    xp = jnp.dot(h, w_ref[...], preferred_element_type=jnp.float32)
    S = _head_selector()
    as8 = jnp.dot(xp * asv_ref[...], S, preferred_element_type=jnp.float32)
    ad8 = jnp.dot(xp * adv_ref[...], S, preferred_element_type=jnp.float32)

    z12 = jnp.zeros((RB, AW - FH - HH), jnp.float32)
    xpa0 = jnp.concatenate([xp[:, :FH], as8[:, :HH], z12], axis=1)
    xpa1 = jnp.concatenate([xp[:, FH:], as8[:, HH:], z12], axis=1)
    xpa_ref[...] = jnp.stack([xpa0, xpa1], axis=0)

    zad = jnp.zeros((RB, 16 - HH), jnp.float32)
    adt_ref[...] = jnp.stack(
        [jnp.concatenate([ad8[:, :HH], zad], axis=1),
         jnp.concatenate([ad8[:, HH:], zad], axis=1)], axis=0)

    @pl.when(i == 0)
    def _():
        acc_ref[...] = jnp.full((8, 16), -1e30, jnp.float32)

    acc_ref[0:1, 0:H] = jnp.maximum(acc_ref[0:1, 0:H],
                                    jnp.max(as8, axis=0, keepdims=True))
    acc_ref[1:2, 0:H] = jnp.maximum(acc_ref[1:2, 0:H],
                                    jnp.max(ad8, axis=0, keepdims=True))

    @pl.when(i == NBLK - 1)
    def _():
        cm = acc_ref[0:1, 0:H] + acc_ref[1:2, 0:H]
        cm = jnp.maximum(cm, 0.2 * cm)
        big = jnp.full((1, 16 - HH), 100.0, jnp.float32)
        row0 = jnp.concatenate([cm[:, :HH], big], axis=1)
        row1 = jnp.concatenate([cm[:, HH:], big], axis=1)
        cmax_ref[...] = jnp.concatenate([row0, row1], axis=0)


def _k_pre(h, w, asv, adv):
    return pl.pallas_call(
        _pre_body,
        grid=(NBLK,),
        in_specs=[
            pl.BlockSpec((RB, HID), lambda i: (i, 0)),
            pl.BlockSpec((HID, HID), lambda i: (0, 0)),
            pl.BlockSpec((1, HID), lambda i: (0, 0)),
            pl.BlockSpec((1, HID), lambda i: (0, 0)),
        ],
        out_specs=[
            pl.BlockSpec((NC, RB, AW), lambda i: (0, i, 0)),
            pl.BlockSpec((NC, RB, 16), lambda i: (0, i, 0)),
            pl.BlockSpec((NC, 16), lambda i: (0, 0)),
        ],
        out_shape=[
            jax.ShapeDtypeStruct((NC, N_PAD, AW), jnp.float32),
            jax.ShapeDtypeStruct((NC, N_PAD, 16), jnp.float32),
            jax.ShapeDtypeStruct((NC, 16), jnp.float32),
        ],
        scratch_shapes=[pltpu.VMEM((8, 16), jnp.float32)],
    )(h, w, asv, adv)


def _denom_expander():
    jj = lax.broadcasted_iota(jnp.int32, (HH, FH), 0)
    ff = lax.broadcasted_iota(jnp.int32, (HH, FH), 1) // DH
    return (jj == ff).astype(jnp.float32)


def _scaled_agg(agg):
    Sx = _denom_expander()
    outs = []
    for c in range(NC):
        num = agg[c, :, :FH]
        den = agg[c, :, FH:FH + HH] + 1e-16
        rec = jnp.dot(1.0 / den, Sx, preferred_element_type=jnp.float32)
        outs.append(num * rec)
    return jnp.concatenate(outs, axis=1)


def _post_common(h_ref, agg_ref, bl_ref, lnw_ref, lnb_ref):
    hmid = h_ref[...] + _scaled_agg(agg_ref[...]) + bl_ref[...]
    m = jnp.mean(hmid, axis=1, keepdims=True)
    d = hmid - m
    v = jnp.mean(d * d, axis=1, keepdims=True)
    hn = d * lax.rsqrt(v + 1e-5) * lnw_ref[...] + lnb_ref[...]
    return jnp.maximum(hn, 0.0)


def _post_body(h_ref, agg_ref, bl_ref, lnw_ref, lnb_ref, o_ref):
    o_ref[...] = _post_common(h_ref, agg_ref, bl_ref, lnw_ref, lnb_ref)


def _k_post(h, agg, bl, lnw, lnb):
    return pl.pallas_call(
        _post_body,
        grid=(NBLK,),
        in_specs=[
            pl.BlockSpec((RB, HID), lambda i: (i, 0)),
            pl.BlockSpec((NC, RB, AW), lambda i: (0, i, 0)),
            pl.BlockSpec((1, HID), lambda i: (0, 0)),
            pl.BlockSpec((1, HID), lambda i: (0, 0)),
            pl.BlockSpec((1, HID), lambda i: (0, 0)),
        ],
        out_specs=pl.BlockSpec((RB, HID), lambda i: (i, 0)),
        out_shape=jax.ShapeDtypeStruct((N_PAD, HID), jnp.float32),
    )(h, agg, bl, lnw, lnb)


def _pool_body(h_ref, agg_ref, bl_ref, lnw_ref, lnb_ref, batch_ref, o_ref,
               sum_ref, cnt_ref):
    i = pl.program_id(0)
    hn = _post_common(h_ref, agg_ref, bl_ref, lnw_ref, lnb_ref)

    bb = batch_ref[0]
    gg = lax.broadcasted_iota(jnp.int32, (G, RB), 0)
    P = (bb == gg).astype(jnp.float32)

    @pl.when(i == 0)
    def _():
        sum_ref[...] = jnp.zeros((G, HID), jnp.float32)
        cnt_ref[...] = jnp.zeros((G, 128), jnp.float32)

    sum_ref[...] += jnp.dot(P, hn, preferred_element_type=jnp.float32)
    cnt_ref[...] += jnp.dot(P, jnp.ones((RB, 128), jnp.float32),
                            preferred_element_type=jnp.float32)

    @pl.when(i == NBLK - 1)
    def _():
        c1 = jnp.maximum(cnt_ref[...], 1.0)
        o_ref[...] = sum_ref[...] / jnp.concatenate([c1, c1], axis=1)


def _k_pool(h, agg, bl, lnw, lnb, batch3):
    return pl.pallas_call(
        _pool_body,
        grid=(NBLK,),
        in_specs=[
            pl.BlockSpec((RB, HID), lambda i: (i, 0)),
            pl.BlockSpec((NC, RB, AW), lambda i: (0, i, 0)),
            pl.BlockSpec((1, HID), lambda i: (0, 0)),
            pl.BlockSpec((1, HID), lambda i: (0, 0)),
            pl.BlockSpec((1, HID), lambda i: (0, 0)),
            pl.BlockSpec((1, 1, RB), lambda i: (i, 0, 0)),
        ],
        out_specs=pl.BlockSpec((G, HID), lambda i: (0, 0)),
        out_shape=jax.ShapeDtypeStruct((G, HID), jnp.float32),
        scratch_shapes=[pltpu.VMEM((G, HID), jnp.float32),
                        pltpu.VMEM((G, 128), jnp.float32)],
    )(h, agg, bl, lnw, lnb, batch3)


_SC_MESH = plsc.VectorSubcoreMesh(
    core_axis_name="c", subcore_axis_name="s", num_cores=NC, num_subcores=NS)

_ROWS_PER = N_PAD // NS


@functools.partial(
    pl.kernel,
    out_type=jax.ShapeDtypeStruct((NC, N_PAD, AW), jnp.float32),
    mesh=_SC_MESH,
    compiler_params=pltpu.CompilerParams(use_tc_tiling_on_sc=False),
    scratch_types=[
        pltpu.VMEM_SHARED((N_PAD, AW), jnp.float32),
        pltpu.VMEM((CB, AW), jnp.float32),
        pltpu.VMEM((CB, AW), jnp.float32),
        pltpu.VMEM((CB, 16), jnp.float32),
        pltpu.VMEM((GRP, CB), jnp.int32),
        pltpu.VMEM((GRP, CB), jnp.int32),
        pltpu.VMEM((GRP, CB), jnp.int32),
        pltpu.VMEM((GRP, CB), jnp.int32),
        pltpu.VMEM((16,), jnp.float32),
        pltpu.SemaphoreType.DMA,
        pltpu.SemaphoreType.DMA,
        pltpu.SemaphoreType.DMA,
        pltpu.SemaphoreType.DMA,
        pltpu.SemaphoreType.DMA,
        pltpu.SemaphoreType.DMA,
        pltpu.SemaphoreType.DMA,
    ],
)
def _sc_edge_kernel(src_hbm, dst_hbm, xpa_hbm, adt_hbm, cmax_hbm, out_hbm,
                    agg_sp, sb0, sb1, dbuf, sg0, sg1, dg0, dg1,
                    cbuf, gs0, gs1, gd, ss0, ss1, gi0, gi1):
    c = lax.axis_index("c")
    s = lax.axis_index("s")
    sbufs = (sb0, sb1)
    gsems = (gs0, gs1)
    ssems = (ss0, ss1)
    sgs = (sg0, sg1)
    dgs = (dg0, dg1)
    gis = (gi0, gi1)

    @pl.loop(0, CB)
    def _zero(e):
        for k in range(AW // 16):
            sb0[e, 16 * k:16 * (k + 1)] = jnp.zeros((16,), jnp.float32)

    full, rem = divmod(_ROWS_PER, CB)
    for j in range(full):
        pltpu.sync_copy(sb0, agg_sp.at[pl.ds(s * _ROWS_PER + j * CB, CB)])
    if rem:
        pltpu.sync_copy(sb0.at[pl.ds(0, rem)],
                        agg_sp.at[pl.ds(s * _ROWS_PER + full * CB, rem)])
    plsc.subcore_barrier()

    pltpu.sync_copy(cmax_hbm.at[c], cbuf)
    C = cbuf[...]

    row0 = s * CHUNKS

    def issue_idx(gi, q):
        pltpu.async_copy(src_hbm.at[pl.ds(row0 + gi * GRP, GRP)], sgs[q],
                         gis[q])
        pltpu.async_copy(dst_hbm.at[pl.ds(row0 + gi * GRP, GRP)], dgs[q],
                         gis[q])

    def wait_idx(gi, q):
        pltpu.make_async_copy(src_hbm.at[pl.ds(row0 + gi * GRP, GRP)],
                              sgs[q], gis[q]).wait()
        pltpu.make_async_copy(dst_hbm.at[pl.ds(row0 + gi * GRP, GRP)],
                              dgs[q], gis[q]).wait()

    def _qr(d):
        return ((d // 2) % 2, d % 2)

    def sidx(d):
        q, r = _qr(d)
        return sgs[q].at[r]

    def didx(d):
        q, r = _qr(d)
        return dgs[q].at[r]

    def issue_gs(d, b):
        pltpu.async_copy(xpa_hbm.at[c].at[sidx(d)], sbufs[b], gsems[b])

    def wait_gs(d, b):
        pltpu.make_async_copy(xpa_hbm.at[c].at[sidx(d)], sbufs[b],
                              gsems[b]).wait()

    def issue_gd(d):
        pltpu.async_copy(adt_hbm.at[c].at[didx(d)], dbuf, gd)

    def wait_gd(d):
        pltpu.make_async_copy(adt_hbm.at[c].at[didx(d)], dbuf, gd).wait()

    def issue_sc(d, b):
        pltpu.async_copy(sbufs[b], agg_sp.at[didx(d)], ssems[b], add=True)

    def wait_sc(d, b):
        pltpu.make_async_copy(sbufs[b], agg_sp.at[didx(d)], ssems[b]).wait()

    def compute_p(sbuf):
        @pl.loop(0, CB)
        def _edge(e):
            arow = sbuf[e, FH:FH + 16]
            drow = dbuf[e, :]
            sm = arow + drow
            sm = jnp.maximum(sm, 0.2 * sm)
            sbuf[e, FH:FH + 16] = jnp.exp(sm - C)

    def compute_w(sbuf):
        @pl.loop(0, CB)
        def _edge(e):
            p16 = sbuf[e, FH:FH + 16]
            for hh_ in range(HH):
                m = jnp.full((16,), p16[hh_], jnp.float32)
                for k2 in range(2):
                    sl = pl.ds(16 * (hh_ * 2 + k2), 16)
                    sbuf[e, sl] = sbuf[e, sl] * m

    issue_idx(0, 0)
    wait_idx(0, 0)
    issue_gd(0)
    issue_gs(0, 0)

    @pl.loop(0, CHUNKS // 4)
    def _super(t):
        i0 = 4 * t
        for p in range(4):
            i = i0 + p
            b = p % 2
            o = 1 - b
            wait_gs(p, b)
            wait_gd(p)
            compute_p(sbufs[b])
            if p in (1, 3):
                pl.when(i + 1 < CHUNKS)(
                    lambda i=i, p=p: wait_idx((i + 1) // GRP, _qr(p + 1)[0]))
            pl.when(i + 1 < CHUNKS)(lambda p=p: issue_gd(p + 1))

            @pl.when(i >= 1)
            def _(p=p, o=o):
                wait_sc(p - 1, o)

            if p == 0:
                pl.when(i + 2 < CHUNKS)(
                    lambda i=i: issue_idx((i + 2) // GRP, 1))
            elif p == 2:
                pl.when(i + 2 < CHUNKS)(
                    lambda i=i: issue_idx((i + 2) // GRP, 0))

            pl.when(i + 1 < CHUNKS)(lambda p=p, o=o: issue_gs(p + 1, o))
            compute_w(sbufs[b])
            issue_sc(p, b)

    wait_sc(3, (CHUNKS - 1) % 2)

    plsc.subcore_barrier()
    for j in range(full):
        r0 = s * _ROWS_PER + j * CB
        pltpu.sync_copy(agg_sp.at[pl.ds(r0, CB)],
                        out_hbm.at[c].at[pl.ds(r0, CB)])
    if rem:
        r0 = s * _ROWS_PER + full * CB
        pltpu.sync_copy(agg_sp.at[pl.ds(r0, rem)],
                        out_hbm.at[c].at[pl.ds(r0, rem)])


def kernel(x, edge_index, batch, W_in, b_in, W_l, a_src, a_dst, b_l, ln_w,
           ln_b):
    f32 = jnp.float32
    x_pad = jnp.zeros((N_PAD, D_IN), f32).at[:N].set(x.astype(f32))
    pad_ids = jnp.full((E_PAD - E,), N, jnp.int32)
    srcp = jnp.concatenate([edge_index[0].astype(jnp.int32), pad_ids]
                           ).reshape(E_PAD // CB, CB)
    dstp = jnp.concatenate([edge_index[1].astype(jnp.int32), pad_ids]
                           ).reshape(E_PAD // CB, CB)
    batch3 = jnp.concatenate(
        [batch.astype(jnp.int32), jnp.full((N_PAD - N,), G, jnp.int32)]
    ).reshape(NBLK, 1, RB)

    b_in2 = b_in.reshape(1, HID).astype(f32)
    asv = a_src.reshape(L, HID).astype(f32)
    adv = a_dst.reshape(L, HID).astype(f32)

    h = _k_in(x_pad, W_in.astype(f32), b_in2)
    out = None
    for l in range(L):
        xpa, adt, cmax = _k_pre(h, W_l[l].astype(f32),
                                asv[l].reshape(1, HID),
                                adv[l].reshape(1, HID))
        agg = _sc_edge_kernel(srcp, dstp, xpa, adt, cmax)
        args = (h, agg, b_l[l].reshape(1, HID).astype(f32),
                ln_w[l].reshape(1, HID).astype(f32),
                ln_b[l].reshape(1, HID).astype(f32))
        if l < L - 1:
            h = _k_post(*args)
        else:
            out = _k_pool(*args, batch3)
    return out


# speedup vs baseline: 44.9013x; 1.2741x over previous
import functools

import jax
import jax.numpy as jnp
from jax import lax
from jax.experimental import pallas as pl
from jax.experimental.pallas import tpu as pltpu
from jax.experimental.pallas import tpu_sc as plsc

N = 10000
E = 320000
D_IN = 128
HID = 256
H = 8
DH = 32
L = 3
G = 64

RB = 128
N_PAD = 10112
NBLK = N_PAD // RB
NC, NS = 2, 16
CB = 128
CHUNKS = 160
GRP = 2
EP_T = CB * CHUNKS
E_PAD = EP_T * NS
HH = H // NC
FH = HID // NC
AW = FH + 16


def _in_body(x_ref, w_ref, b_ref, h_ref):
    h_ref[...] = (
        jnp.dot(x_ref[...], w_ref[...], preferred_element_type=jnp.float32)
        + b_ref[...]
    )


def _k_in(x, w, b):
    return pl.pallas_call(
        _in_body,
        grid=(NBLK,),
        in_specs=[
            pl.BlockSpec((RB, D_IN), lambda i: (i, 0)),
            pl.BlockSpec((D_IN, HID), lambda i: (0, 0)),
            pl.BlockSpec((1, HID), lambda i: (0, 0)),
        ],
        out_specs=pl.BlockSpec((RB, HID), lambda i: (i, 0)),
        out_shape=jax.ShapeDtypeStruct((N_PAD, HID), jnp.float32),
    )(x, w, b)


def _head_selector():
    kk = lax.broadcasted_iota(jnp.int32, (HID, H), 0) // DH
    hh = lax.broadcasted_iota(jnp.int32, (HID, H), 1)
    return (kk == hh).astype(jnp.float32)


def _pre_body(h_ref, w_ref, asv_ref, adv_ref, xpa_ref, adt_ref, cmax_ref,
              acc_ref):
    i = pl.program_id(0)
    h = h_ref[...]
    xp = jnp.dot(h, w_ref[...], preferred_element_type=jnp.float32)
    S = _head_selector()
    as8 = jnp.dot(xp * asv_ref[...], S, preferred_element_type=jnp.float32)
    ad8 = jnp.dot(xp * adv_ref[...], S, preferred_element_type=jnp.float32)

    z12 = jnp.zeros((RB, AW - FH - HH), jnp.float32)
    xpa0 = jnp.concatenate([xp[:, :FH], as8[:, :HH], z12], axis=1)
    xpa1 = jnp.concatenate([xp[:, FH:], as8[:, HH:], z12], axis=1)
    xpa_ref[...] = jnp.stack([xpa0, xpa1], axis=0)

    zad = jnp.zeros((RB, 16 - HH), jnp.float32)
    adt_ref[...] = jnp.stack(
        [jnp.concatenate([ad8[:, :HH], zad], axis=1),
         jnp.concatenate([ad8[:, HH:], zad], axis=1)], axis=0)

    @pl.when(i == 0)
    def _():
        acc_ref[...] = jnp.full((8, 16), -1e30, jnp.float32)

    acc_ref[0:1, 0:H] = jnp.maximum(acc_ref[0:1, 0:H],
                                    jnp.max(as8, axis=0, keepdims=True))
    acc_ref[1:2, 0:H] = jnp.maximum(acc_ref[1:2, 0:H],
                                    jnp.max(ad8, axis=0, keepdims=True))

    @pl.when(i == NBLK - 1)
    def _():
        cm = acc_ref[0:1, 0:H] + acc_ref[1:2, 0:H]
        cm = jnp.maximum(cm, 0.2 * cm)
        big = jnp.full((1, 16 - HH), 100.0, jnp.float32)
        row0 = jnp.concatenate([cm[:, :HH], big], axis=1)
        row1 = jnp.concatenate([cm[:, HH:], big], axis=1)
        cmax_ref[...] = jnp.concatenate([row0, row1], axis=0)


def _k_pre(h, w, asv, adv):
    return pl.pallas_call(
        _pre_body,
        grid=(NBLK,),
        in_specs=[
            pl.BlockSpec((RB, HID), lambda i: (i, 0)),
            pl.BlockSpec((HID, HID), lambda i: (0, 0)),
            pl.BlockSpec((1, HID), lambda i: (0, 0)),
            pl.BlockSpec((1, HID), lambda i: (0, 0)),
        ],
        out_specs=[
            pl.BlockSpec((NC, RB, AW), lambda i: (0, i, 0)),
            pl.BlockSpec((NC, RB, 16), lambda i: (0, i, 0)),
            pl.BlockSpec((NC, 16), lambda i: (0, 0)),
        ],
        out_shape=[
            jax.ShapeDtypeStruct((NC, N_PAD, AW), jnp.float32),
            jax.ShapeDtypeStruct((NC, N_PAD, 16), jnp.float32),
            jax.ShapeDtypeStruct((NC, 16), jnp.float32),
        ],
        scratch_shapes=[pltpu.VMEM((8, 16), jnp.float32)],
    )(h, w, asv, adv)


def _denom_expander():
    jj = lax.broadcasted_iota(jnp.int32, (HH, FH), 0)
    ff = lax.broadcasted_iota(jnp.int32, (HH, FH), 1) // DH
    return (jj == ff).astype(jnp.float32)


def _scaled_agg(agg):
    Sx = _denom_expander()
    outs = []
    for c in range(NC):
        num = agg[c, :, :FH]
        den = agg[c, :, FH:FH + HH] + 1e-16
        rec = jnp.dot(1.0 / den, Sx, preferred_element_type=jnp.float32)
        outs.append(num * rec)
    return jnp.concatenate(outs, axis=1)


def _post_common(h_ref, agg_ref, bl_ref, lnw_ref, lnb_ref):
    hmid = h_ref[...] + _scaled_agg(agg_ref[...]) + bl_ref[...]
    m = jnp.mean(hmid, axis=1, keepdims=True)
    d = hmid - m
    v = jnp.mean(d * d, axis=1, keepdims=True)
    hn = d * lax.rsqrt(v + 1e-5) * lnw_ref[...] + lnb_ref[...]
    return jnp.maximum(hn, 0.0)


def _post_body(h_ref, agg_ref, bl_ref, lnw_ref, lnb_ref, o_ref):
    o_ref[...] = _post_common(h_ref, agg_ref, bl_ref, lnw_ref, lnb_ref)


def _k_post(h, agg, bl, lnw, lnb):
    return pl.pallas_call(
        _post_body,
        grid=(NBLK,),
        in_specs=[
            pl.BlockSpec((RB, HID), lambda i: (i, 0)),
            pl.BlockSpec((NC, RB, AW), lambda i: (0, i, 0)),
            pl.BlockSpec((1, HID), lambda i: (0, 0)),
            pl.BlockSpec((1, HID), lambda i: (0, 0)),
            pl.BlockSpec((1, HID), lambda i: (0, 0)),
        ],
        out_specs=pl.BlockSpec((RB, HID), lambda i: (i, 0)),
        out_shape=jax.ShapeDtypeStruct((N_PAD, HID), jnp.float32),
    )(h, agg, bl, lnw, lnb)


def _pool_body(h_ref, agg_ref, bl_ref, lnw_ref, lnb_ref, batch_ref, o_ref,
               sum_ref, cnt_ref):
    i = pl.program_id(0)
    hn = _post_common(h_ref, agg_ref, bl_ref, lnw_ref, lnb_ref)

    bb = batch_ref[0]
    gg = lax.broadcasted_iota(jnp.int32, (G, RB), 0)
    P = (bb == gg).astype(jnp.float32)

    @pl.when(i == 0)
    def _():
        sum_ref[...] = jnp.zeros((G, HID), jnp.float32)
        cnt_ref[...] = jnp.zeros((G, 128), jnp.float32)

    sum_ref[...] += jnp.dot(P, hn, preferred_element_type=jnp.float32)
    cnt_ref[...] += jnp.dot(P, jnp.ones((RB, 128), jnp.float32),
                            preferred_element_type=jnp.float32)

    @pl.when(i == NBLK - 1)
    def _():
        c1 = jnp.maximum(cnt_ref[...], 1.0)
        o_ref[...] = sum_ref[...] / jnp.concatenate([c1, c1], axis=1)


def _k_pool(h, agg, bl, lnw, lnb, batch3):
    return pl.pallas_call(
        _pool_body,
        grid=(NBLK,),
        in_specs=[
            pl.BlockSpec((RB, HID), lambda i: (i, 0)),
            pl.BlockSpec((NC, RB, AW), lambda i: (0, i, 0)),
            pl.BlockSpec((1, HID), lambda i: (0, 0)),
            pl.BlockSpec((1, HID), lambda i: (0, 0)),
            pl.BlockSpec((1, HID), lambda i: (0, 0)),
            pl.BlockSpec((1, 1, RB), lambda i: (i, 0, 0)),
        ],
        out_specs=pl.BlockSpec((G, HID), lambda i: (0, 0)),
        out_shape=jax.ShapeDtypeStruct((G, HID), jnp.float32),
        scratch_shapes=[pltpu.VMEM((G, HID), jnp.float32),
                        pltpu.VMEM((G, 128), jnp.float32)],
    )(h, agg, bl, lnw, lnb, batch3)


_SC_MESH = plsc.VectorSubcoreMesh(
    core_axis_name="c", subcore_axis_name="s", num_cores=NC, num_subcores=NS)

_ROWS_PER = N_PAD // NS


@functools.partial(
    pl.kernel,
    out_type=jax.ShapeDtypeStruct((NC, N_PAD, AW), jnp.float32),
    mesh=_SC_MESH,
    compiler_params=pltpu.CompilerParams(use_tc_tiling_on_sc=False),
    scratch_types=[
        pltpu.VMEM_SHARED((N_PAD, AW), jnp.float32),
        pltpu.VMEM((CB, AW), jnp.float32),
        pltpu.VMEM((CB, AW), jnp.float32),
        pltpu.VMEM((CB, 16), jnp.float32),
        pltpu.VMEM((GRP, CB), jnp.int32),
        pltpu.VMEM((GRP, CB), jnp.int32),
        pltpu.VMEM((GRP, CB), jnp.int32),
        pltpu.VMEM((GRP, CB), jnp.int32),
        pltpu.VMEM((16,), jnp.float32),
        pltpu.SemaphoreType.DMA,
        pltpu.SemaphoreType.DMA,
        pltpu.SemaphoreType.DMA,
        pltpu.SemaphoreType.DMA,
        pltpu.SemaphoreType.DMA,
        pltpu.SemaphoreType.DMA,
        pltpu.SemaphoreType.DMA,
    ],
)
def _sc_edge_kernel(src_hbm, dst_hbm, xpa_hbm, adt_hbm, cmax_hbm, out_hbm,
                    agg_sp, sb0, sb1, dbuf, sg0, sg1, dg0, dg1,
                    cbuf, gs0, gs1, gd, ss0, ss1, gi0, gi1):
    c = lax.axis_index("c")
    s = lax.axis_index("s")
    sbufs = (sb0, sb1)
    gsems = (gs0, gs1)
    ssems = (ss0, ss1)
    sgs = (sg0, sg1)
    dgs = (dg0, dg1)
    gis = (gi0, gi1)

    @pl.loop(0, CB)
    def _zero(e):
        for k in range(AW // 16):
            sb0[e, 16 * k:16 * (k + 1)] = jnp.zeros((16,), jnp.float32)

    full, rem = divmod(_ROWS_PER, CB)
    for j in range(full):
        pltpu.sync_copy(sb0, agg_sp.at[pl.ds(s * _ROWS_PER + j * CB, CB)])
    if rem:
        pltpu.sync_copy(sb0.at[pl.ds(0, rem)],
                        agg_sp.at[pl.ds(s * _ROWS_PER + full * CB, rem)])
    plsc.subcore_barrier()

    pltpu.sync_copy(cmax_hbm.at[c], cbuf)
    C = cbuf[...]

    row0 = s * CHUNKS

    def issue_idx(gi, q):
        pltpu.async_copy(src_hbm.at[pl.ds(row0 + gi * GRP, GRP)], sgs[q],
                         gis[q])
        pltpu.async_copy(dst_hbm.at[pl.ds(row0 + gi * GRP, GRP)], dgs[q],
                         gis[q])

    def wait_idx(gi, q):
        pltpu.make_async_copy(src_hbm.at[pl.ds(row0 + gi * GRP, GRP)],
                              sgs[q], gis[q]).wait()
        pltpu.make_async_copy(dst_hbm.at[pl.ds(row0 + gi * GRP, GRP)],
                              dgs[q], gis[q]).wait()

    def _qr(d):
        return ((d // 2) % 2, d % 2)

    def sidx(d):
        q, r = _qr(d)
        return sgs[q].at[r]

    def didx(d):
        q, r = _qr(d)
        return dgs[q].at[r]

    def issue_gs(d, b):
        pltpu.async_copy(xpa_hbm.at[c].at[sidx(d)], sbufs[b], gsems[b])

    def wait_gs(d, b):
        pltpu.make_async_copy(xpa_hbm.at[c].at[sidx(d)], sbufs[b],
                              gsems[b]).wait()

    def issue_gd(d):
        pltpu.async_copy(adt_hbm.at[c].at[didx(d)], dbuf, gd)

    def wait_gd(d):
        pltpu.make_async_copy(adt_hbm.at[c].at[didx(d)], dbuf, gd).wait()

    def issue_sc(d, b):
        pltpu.async_copy(sbufs[b], agg_sp.at[didx(d)], ssems[b], add=True)

    def wait_sc(d, b):
        pltpu.make_async_copy(sbufs[b], agg_sp.at[didx(d)], ssems[b]).wait()

    def compute_p(sbuf):
        @plsc.parallel_loop(0, CB, unroll=8)
        def _edge(e):
            arow = sbuf[e, FH:FH + 16]
            drow = dbuf[e, :]
            sm = arow + drow
            sm = jnp.maximum(sm, 0.2 * sm)
            sbuf[e, FH:FH + 16] = jnp.exp(sm - C)

    def compute_w(sbuf):
        @plsc.parallel_loop(0, CB, unroll=4)
        def _edge(e):
            p16 = sbuf[e, FH:FH + 16]
            for hh_ in range(HH):
                m = jnp.full((16,), p16[hh_], jnp.float32)
                for k2 in range(2):
                    sl = pl.ds(16 * (hh_ * 2 + k2), 16)
                    sbuf[e, sl] = sbuf[e, sl] * m

    issue_idx(0, 0)
    wait_idx(0, 0)
    issue_gd(0)
    issue_gs(0, 0)

    @pl.loop(0, CHUNKS // 4)
    def _super(t):
        i0 = 4 * t
        for p in range(4):
            i = i0 + p
            b = p % 2
            o = 1 - b
            wait_gs(p, b)
            wait_gd(p)
            compute_p(sbufs[b])
            if p in (1, 3):
                pl.when(i + 1 < CHUNKS)(
                    lambda i=i, p=p: wait_idx((i + 1) // GRP, _qr(p + 1)[0]))
            pl.when(i + 1 < CHUNKS)(lambda p=p: issue_gd(p + 1))

            @pl.when(i >= 1)
            def _(p=p, o=o):
                wait_sc(p - 1, o)

            if p == 0:
                pl.when(i + 2 < CHUNKS)(
                    lambda i=i: issue_idx((i + 2) // GRP, 1))
            elif p == 2:
                pl.when(i + 2 < CHUNKS)(
                    lambda i=i: issue_idx((i + 2) // GRP, 0))

            pl.when(i + 1 < CHUNKS)(lambda p=p, o=o: issue_gs(p + 1, o))
            compute_w(sbufs[b])
            issue_sc(p, b)

    wait_sc(3, (CHUNKS - 1) % 2)

    plsc.subcore_barrier()
    for j in range(full):
        r0 = s * _ROWS_PER + j * CB
        pltpu.sync_copy(agg_sp.at[pl.ds(r0, CB)],
                        out_hbm.at[c].at[pl.ds(r0, CB)])
    if rem:
        r0 = s * _ROWS_PER + full * CB
        pltpu.sync_copy(agg_sp.at[pl.ds(r0, rem)],
                        out_hbm.at[c].at[pl.ds(r0, rem)])


def kernel(x, edge_index, batch, W_in, b_in, W_l, a_src, a_dst, b_l, ln_w,
           ln_b):
    f32 = jnp.float32
    x_pad = jnp.zeros((N_PAD, D_IN), f32).at[:N].set(x.astype(f32))
    pad_ids = jnp.full((E_PAD - E,), N, jnp.int32)
    srcp = jnp.concatenate([edge_index[0].astype(jnp.int32), pad_ids]
                           ).reshape(E_PAD // CB, CB)
    dstp = jnp.concatenate([edge_index[1].astype(jnp.int32), pad_ids]
                           ).reshape(E_PAD // CB, CB)
    batch3 = jnp.concatenate(
        [batch.astype(jnp.int32), jnp.full((N_PAD - N,), G, jnp.int32)]
    ).reshape(NBLK, 1, RB)

    b_in2 = b_in.reshape(1, HID).astype(f32)
    asv = a_src.reshape(L, HID).astype(f32)
    adv = a_dst.reshape(L, HID).astype(f32)

    h = _k_in(x_pad, W_in.astype(f32), b_in2)
    out = None
    for l in range(L):
        xpa, adt, cmax = _k_pre(h, W_l[l].astype(f32),
                                asv[l].reshape(1, HID),
                                adv[l].reshape(1, HID))
        agg = _sc_edge_kernel(srcp, dstp, xpa, adt, cmax)
        args = (h, agg, b_l[l].reshape(1, HID).astype(f32),
                ln_w[l].reshape(1, HID).astype(f32),
                ln_b[l].reshape(1, HID).astype(f32))
        if l < L - 1:
            h = _k_post(*args)
        else:
            out = _k_pool(*args, batch3)
    return out


# speedup vs baseline: 46.3564x; 1.0324x over previous
import functools

import jax
import jax.numpy as jnp
from jax import lax
from jax.experimental import pallas as pl
from jax.experimental.pallas import tpu as pltpu
from jax.experimental.pallas import tpu_sc as plsc

N = 10000
E = 320000
D_IN = 128
HID = 256
H = 8
DH = 32
L = 3
G = 64

RB = 128
N_PAD = 10112
NBLK = N_PAD // RB
NC, NS = 2, 16
CB = 128
CHUNKS = 160
GRP = 2
EP_T = CB * CHUNKS
E_PAD = EP_T * NS
HH = H // NC
FH = HID // NC
AW = FH + 16


def _in_body(x_ref, w_ref, b_ref, h_ref):
    h_ref[...] = (
        jnp.dot(x_ref[...], w_ref[...], preferred_element_type=jnp.float32)
        + b_ref[...]
    )


def _k_in(x, w, b):
    return pl.pallas_call(
        _in_body,
        grid=(NBLK,),
        in_specs=[
            pl.BlockSpec((RB, D_IN), lambda i: (i, 0)),
            pl.BlockSpec((D_IN, HID), lambda i: (0, 0)),
            pl.BlockSpec((1, HID), lambda i: (0, 0)),
        ],
        out_specs=pl.BlockSpec((RB, HID), lambda i: (i, 0)),
        out_shape=jax.ShapeDtypeStruct((N_PAD, HID), jnp.float32),
    )(x, w, b)


def _head_selector():
    kk = lax.broadcasted_iota(jnp.int32, (HID, H), 0) // DH
    hh = lax.broadcasted_iota(jnp.int32, (HID, H), 1)
    return (kk == hh).astype(jnp.float32)


def _pre_body(h_ref, w_ref, asv_ref, adv_ref, xpa_ref, adt_ref, cmax_ref,
              acc_ref):
    i = pl.program_id(0)
    h = h_ref[...]
    xp = jnp.dot(h, w_ref[...], preferred_element_type=jnp.float32)
    S = _head_selector()
    as8 = jnp.dot(xp * asv_ref[...], S, preferred_element_type=jnp.float32)
    ad8 = jnp.dot(xp * adv_ref[...], S, preferred_element_type=jnp.float32)

    z12 = jnp.zeros((RB, AW - FH - HH), jnp.float32)
    xpa0 = jnp.concatenate([xp[:, :FH], as8[:, :HH], z12], axis=1)
    xpa1 = jnp.concatenate([xp[:, FH:], as8[:, HH:], z12], axis=1)
    xpa_ref[...] = jnp.stack([xpa0, xpa1], axis=0)

    zad = jnp.zeros((RB, 16 - HH), jnp.float32)
    adt_ref[...] = jnp.stack(
        [jnp.concatenate([ad8[:, :HH], zad], axis=1),
         jnp.concatenate([ad8[:, HH:], zad], axis=1)], axis=0)

    @pl.when(i == 0)
    def _():
        acc_ref[...] = jnp.full((8, 16), -1e30, jnp.float32)

    acc_ref[0:1, 0:H] = jnp.maximum(acc_ref[0:1, 0:H],
                                    jnp.max(as8, axis=0, keepdims=True))
    acc_ref[1:2, 0:H] = jnp.maximum(acc_ref[1:2, 0:H],
                                    jnp.max(ad8, axis=0, keepdims=True))

    @pl.when(i == NBLK - 1)
    def _():
        cm = acc_ref[0:1, 0:H] + acc_ref[1:2, 0:H]
        cm = jnp.maximum(cm, 0.2 * cm)
        big = jnp.full((1, 16 - HH), 100.0, jnp.float32)
        row0 = jnp.concatenate([cm[:, :HH], big], axis=1)
        row1 = jnp.concatenate([cm[:, HH:], big], axis=1)
        cmax_ref[...] = jnp.concatenate([row0, row1], axis=0)


def _k_pre(h, w, asv, adv):
    return pl.pallas_call(
        _pre_body,
        grid=(NBLK,),
        in_specs=[
            pl.BlockSpec((RB, HID), lambda i: (i, 0)),
            pl.BlockSpec((HID, HID), lambda i: (0, 0)),
            pl.BlockSpec((1, HID), lambda i: (0, 0)),
            pl.BlockSpec((1, HID), lambda i: (0, 0)),
        ],
        out_specs=[
            pl.BlockSpec((NC, RB, AW), lambda i: (0, i, 0)),
            pl.BlockSpec((NC, RB, 16), lambda i: (0, i, 0)),
            pl.BlockSpec((NC, 16), lambda i: (0, 0)),
        ],
        out_shape=[
            jax.ShapeDtypeStruct((NC, N_PAD, AW), jnp.float32),
            jax.ShapeDtypeStruct((NC, N_PAD, 16), jnp.float32),
            jax.ShapeDtypeStruct((NC, 16), jnp.float32),
        ],
        scratch_shapes=[pltpu.VMEM((8, 16), jnp.float32)],
    )(h, w, asv, adv)


def _denom_expander():
    jj = lax.broadcasted_iota(jnp.int32, (HH, FH), 0)
    ff = lax.broadcasted_iota(jnp.int32, (HH, FH), 1) // DH
    return (jj == ff).astype(jnp.float32)


def _scaled_agg(agg):
    Sx = _denom_expander()
    outs = []
    for c in range(NC):
        num = agg[c, :, :FH]
        den = agg[c, :, FH:FH + HH] + 1e-16
        rec = jnp.dot(1.0 / den, Sx, preferred_element_type=jnp.float32)
        outs.append(num * rec)
    return jnp.concatenate(outs, axis=1)


def _post_common(h_ref, agg_ref, bl_ref, lnw_ref, lnb_ref):
    hmid = h_ref[...] + _scaled_agg(agg_ref[...]) + bl_ref[...]
    m = jnp.mean(hmid, axis=1, keepdims=True)
    d = hmid - m
    v = jnp.mean(d * d, axis=1, keepdims=True)
    hn = d * lax.rsqrt(v + 1e-5) * lnw_ref[...] + lnb_ref[...]
    return jnp.maximum(hn, 0.0)


def _post_body(h_ref, agg_ref, bl_ref, lnw_ref, lnb_ref, o_ref):
    o_ref[...] = _post_common(h_ref, agg_ref, bl_ref, lnw_ref, lnb_ref)


def _k_post(h, agg, bl, lnw, lnb):
    return pl.pallas_call(
        _post_body,
        grid=(NBLK,),
        in_specs=[
            pl.BlockSpec((RB, HID), lambda i: (i, 0)),
            pl.BlockSpec((NC, RB, AW), lambda i: (0, i, 0)),
            pl.BlockSpec((1, HID), lambda i: (0, 0)),
            pl.BlockSpec((1, HID), lambda i: (0, 0)),
            pl.BlockSpec((1, HID), lambda i: (0, 0)),
        ],
        out_specs=pl.BlockSpec((RB, HID), lambda i: (i, 0)),
        out_shape=jax.ShapeDtypeStruct((N_PAD, HID), jnp.float32),
    )(h, agg, bl, lnw, lnb)


def _pool_body(h_ref, agg_ref, bl_ref, lnw_ref, lnb_ref, batch_ref, o_ref,
               sum_ref, cnt_ref):
    i = pl.program_id(0)
    hn = _post_common(h_ref, agg_ref, bl_ref, lnw_ref, lnb_ref)

    bb = batch_ref[0]
    gg = lax.broadcasted_iota(jnp.int32, (G, RB), 0)
    P = (bb == gg).astype(jnp.float32)

    @pl.when(i == 0)
    def _():
        sum_ref[...] = jnp.zeros((G, HID), jnp.float32)
        cnt_ref[...] = jnp.zeros((G, 128), jnp.float32)

    sum_ref[...] += jnp.dot(P, hn, preferred_element_type=jnp.float32)
    cnt_ref[...] += jnp.dot(P, jnp.ones((RB, 128), jnp.float32),
                            preferred_element_type=jnp.float32)

    @pl.when(i == NBLK - 1)
    def _():
        c1 = jnp.maximum(cnt_ref[...], 1.0)
        o_ref[...] = sum_ref[...] / jnp.concatenate([c1, c1], axis=1)


def _k_pool(h, agg, bl, lnw, lnb, batch3):
    return pl.pallas_call(
        _pool_body,
        grid=(NBLK,),
        in_specs=[
            pl.BlockSpec((RB, HID), lambda i: (i, 0)),
            pl.BlockSpec((NC, RB, AW), lambda i: (0, i, 0)),
            pl.BlockSpec((1, HID), lambda i: (0, 0)),
            pl.BlockSpec((1, HID), lambda i: (0, 0)),
            pl.BlockSpec((1, HID), lambda i: (0, 0)),
            pl.BlockSpec((1, 1, RB), lambda i: (i, 0, 0)),
        ],
        out_specs=pl.BlockSpec((G, HID), lambda i: (0, 0)),
        out_shape=jax.ShapeDtypeStruct((G, HID), jnp.float32),
        scratch_shapes=[pltpu.VMEM((G, HID), jnp.float32),
                        pltpu.VMEM((G, 128), jnp.float32)],
    )(h, agg, bl, lnw, lnb, batch3)


_SC_MESH = plsc.VectorSubcoreMesh(
    core_axis_name="c", subcore_axis_name="s", num_cores=NC, num_subcores=NS)

_ROWS_PER = N_PAD // NS


@functools.partial(
    pl.kernel,
    out_type=jax.ShapeDtypeStruct((NC, N_PAD, AW), jnp.float32),
    mesh=_SC_MESH,
    compiler_params=pltpu.CompilerParams(use_tc_tiling_on_sc=False),
    scratch_types=[
        pltpu.VMEM_SHARED((N_PAD, AW), jnp.float32),
        pltpu.VMEM((CB, AW), jnp.float32),
        pltpu.VMEM((CB, AW), jnp.float32),
        pltpu.VMEM((CB, 16), jnp.float32),
        pltpu.VMEM((GRP, CB), jnp.int32),
        pltpu.VMEM((GRP, CB), jnp.int32),
        pltpu.VMEM((GRP, CB), jnp.int32),
        pltpu.VMEM((GRP, CB), jnp.int32),
        pltpu.VMEM((16,), jnp.float32),
        pltpu.SemaphoreType.DMA,
        pltpu.SemaphoreType.DMA,
        pltpu.SemaphoreType.DMA,
        pltpu.SemaphoreType.DMA,
        pltpu.SemaphoreType.DMA,
        pltpu.SemaphoreType.DMA,
        pltpu.SemaphoreType.DMA,
    ],
)
def _sc_edge_kernel(src_hbm, dst_hbm, xpa_hbm, adt_hbm, cmax_hbm, out_hbm,
                    agg_sp, sb0, sb1, dbuf, sg0, sg1, dg0, dg1,
                    cbuf, gs0, gs1, gd, ss0, ss1, gi0, gi1):
    c = lax.axis_index("c")
    s = lax.axis_index("s")
    sbufs = (sb0, sb1)
    gsems = (gs0, gs1)
    ssems = (ss0, ss1)
    sgs = (sg0, sg1)
    dgs = (dg0, dg1)
    gis = (gi0, gi1)

    @pl.loop(0, CB)
    def _zero(e):
        for k in range(AW // 16):
            sb0[e, 16 * k:16 * (k + 1)] = jnp.zeros((16,), jnp.float32)

    full, rem = divmod(_ROWS_PER, CB)
    for j in range(full):
        pltpu.sync_copy(sb0, agg_sp.at[pl.ds(s * _ROWS_PER + j * CB, CB)])
    if rem:
        pltpu.sync_copy(sb0.at[pl.ds(0, rem)],
                        agg_sp.at[pl.ds(s * _ROWS_PER + full * CB, rem)])
    plsc.subcore_barrier()

    pltpu.sync_copy(cmax_hbm.at[c], cbuf)
    C = cbuf[...]

    row0 = s * CHUNKS

    def issue_idx(gi, q):
        pltpu.async_copy(src_hbm.at[pl.ds(row0 + gi * GRP, GRP)], sgs[q],
                         gis[q])
        pltpu.async_copy(dst_hbm.at[pl.ds(row0 + gi * GRP, GRP)], dgs[q],
                         gis[q])

    def wait_idx(gi, q):
        pltpu.make_async_copy(src_hbm.at[pl.ds(row0 + gi * GRP, GRP)],
                              sgs[q], gis[q]).wait()
        pltpu.make_async_copy(dst_hbm.at[pl.ds(row0 + gi * GRP, GRP)],
                              dgs[q], gis[q]).wait()

    def _qr(d):
        return ((d // 2) % 2, d % 2)

    def sidx(d):
        q, r = _qr(d)
        return sgs[q].at[r]

    def didx(d):
        q, r = _qr(d)
        return dgs[q].at[r]

    NSPLIT = 4
    SB = CB // NSPLIT

    def _sidx_part(d, j):
        q, r = _qr(d)
        return sgs[q].at[r, pl.ds(j * SB, SB)]

    def issue_gs(d, b):
        for j in range(NSPLIT):
            pltpu.async_copy(xpa_hbm.at[c].at[_sidx_part(d, j)],
                             sbufs[b].at[pl.ds(j * SB, SB)], gsems[b])

    def wait_gs(d, b):
        for j in range(NSPLIT):
            pltpu.make_async_copy(xpa_hbm.at[c].at[_sidx_part(d, j)],
                                  sbufs[b].at[pl.ds(j * SB, SB)],
                                  gsems[b]).wait()

    def issue_gd(d):
        pltpu.async_copy(adt_hbm.at[c].at[didx(d)], dbuf, gd)

    def wait_gd(d):
        pltpu.make_async_copy(adt_hbm.at[c].at[didx(d)], dbuf, gd).wait()

    def issue_sc(d, b):
        pltpu.async_copy(sbufs[b], agg_sp.at[didx(d)], ssems[b], add=True)

    def wait_sc(d, b):
        pltpu.make_async_copy(sbufs[b], agg_sp.at[didx(d)], ssems[b]).wait()

    def compute_p(sbuf):
        @plsc.parallel_loop(0, CB, unroll=8)
        def _edge(e):
            arow = sbuf[e, FH:FH + 16]
            drow = dbuf[e, :]
            sm = arow + drow
            sm = jnp.maximum(sm, 0.2 * sm)
            sbuf[e, FH:FH + 16] = jnp.exp(sm - C)

    def compute_w(sbuf):
        @plsc.parallel_loop(0, CB, unroll=4)
        def _edge(e):
            p16 = sbuf[e, FH:FH + 16]
            for hh_ in range(HH):
                m = jnp.full((16,), p16[hh_], jnp.float32)
                for k2 in range(2):
                    sl = pl.ds(16 * (hh_ * 2 + k2), 16)
                    sbuf[e, sl] = sbuf[e, sl] * m

    issue_idx(0, 0)
    wait_idx(0, 0)
    issue_gd(0)
    issue_gs(0, 0)

    @pl.loop(0, CHUNKS // 4)
    def _super(t):
        i0 = 4 * t
        for p in range(4):
            i = i0 + p
            b = p % 2
            o = 1 - b
            wait_gs(p, b)
            wait_gd(p)

            if p in (1, 3):
                pl.when(i + 1 < CHUNKS)(
                    lambda i=i, p=p: wait_idx((i + 1) // GRP, _qr(p + 1)[0]))
            pl.when(i + 1 < CHUNKS)(lambda p=p: issue_gd(p + 1))

            @pl.when(i >= 1)
            def _(p=p, o=o):
                wait_sc(p - 1, o)

            if p == 0:
                pl.when(i + 2 < CHUNKS)(
                    lambda i=i: issue_idx((i + 2) // GRP, 1))
            elif p == 2:
                pl.when(i + 2 < CHUNKS)(
                    lambda i=i: issue_idx((i + 2) // GRP, 0))

            pl.when(i + 1 < CHUNKS)(lambda p=p, o=o: issue_gs(p + 1, o))
            issue_sc(p, b)

    wait_sc(3, (CHUNKS - 1) % 2)

    plsc.subcore_barrier()
    for j in range(full):
        r0 = s * _ROWS_PER + j * CB
        pltpu.sync_copy(agg_sp.at[pl.ds(r0, CB)],
                        out_hbm.at[c].at[pl.ds(r0, CB)])
    if rem:
        r0 = s * _ROWS_PER + full * CB
        pltpu.sync_copy(agg_sp.at[pl.ds(r0, rem)],
                        out_hbm.at[c].at[pl.ds(r0, rem)])


def kernel(x, edge_index, batch, W_in, b_in, W_l, a_src, a_dst, b_l, ln_w,
           ln_b):
    f32 = jnp.float32
    x_pad = jnp.zeros((N_PAD, D_IN), f32).at[:N].set(x.astype(f32))
    pad_ids = jnp.full((E_PAD - E,), N, jnp.int32)
    srcp = jnp.concatenate([edge_index[0].astype(jnp.int32), pad_ids]
                           ).reshape(E_PAD // CB, CB)
    dstp = jnp.concatenate([edge_index[1].astype(jnp.int32), pad_ids]
                           ).reshape(E_PAD // CB, CB)
    batch3 = jnp.concatenate(
        [batch.astype(jnp.int32), jnp.full((N_PAD - N,), G, jnp.int32)]
    ).reshape(NBLK, 1, RB)

    b_in2 = b_in.reshape(1, HID).astype(f32)
    asv = a_src.reshape(L, HID).astype(f32)
    adv = a_dst.reshape(L, HID).astype(f32)

    h = _k_in(x_pad, W_in.astype(f32), b_in2)
    out = None
    for l in range(L):
        xpa, adt, cmax = _k_pre(h, W_l[l].astype(f32),
                                asv[l].reshape(1, HID),
                                adv[l].reshape(1, HID))
        agg = _sc_edge_kernel(srcp, dstp, xpa, adt, cmax)
        args = (h, agg, b_l[l].reshape(1, HID).astype(f32),
                ln_w[l].reshape(1, HID).astype(f32),
                ln_b[l].reshape(1, HID).astype(f32))
        if l < L - 1:
            h = _k_post(*args)
        else:
            out = _k_pool(*args, batch3)
    return out


# speedup vs baseline: 63.7606x; 1.3754x over previous
import functools

import jax
import jax.numpy as jnp
from jax import lax
from jax.experimental import pallas as pl
from jax.experimental.pallas import tpu as pltpu
from jax.experimental.pallas import tpu_sc as plsc

N = 10000
E = 320000
D_IN = 128
HID = 256
H = 8
DH = 32
L = 3
G = 64

RB = 128
N_PAD = 10112
NBLK = N_PAD // RB
NC, NS = 2, 16
CB = 128
CHUNKS = 160
GRP = 2
EP_T = CB * CHUNKS
E_PAD = EP_T * NS
HH = H // NC
FH = HID // NC
AW = FH + 16
XW = FH // 2 + 8
DEN0 = FH + 8


def _in_body(x_ref, w_ref, b_ref, h_ref):
    h_ref[...] = (
        jnp.dot(x_ref[...], w_ref[...], preferred_element_type=jnp.float32)
        + b_ref[...]
    )


def _k_in(x, w, b):
    return pl.pallas_call(
        _in_body,
        grid=(NBLK,),
        in_specs=[
            pl.BlockSpec((RB, D_IN), lambda i: (i, 0)),
            pl.BlockSpec((D_IN, HID), lambda i: (0, 0)),
            pl.BlockSpec((1, HID), lambda i: (0, 0)),
        ],
        out_specs=pl.BlockSpec((RB, HID), lambda i: (i, 0)),
        out_shape=jax.ShapeDtypeStruct((N_PAD, HID), jnp.float32),
    )(x, w, b)


def _half_selector():
    ww = lax.broadcasted_iota(jnp.int32, (FH // 2, HH), 0) // 16
    tt = lax.broadcasted_iota(jnp.int32, (FH // 2, HH), 1)
    return (ww == tt).astype(jnp.float32)


def _bf16_bits(x):
    xr = x.astype(jnp.bfloat16).astype(jnp.float32)
    return lax.bitcast_convert_type(xr, jnp.int32)


def _pre_body(h_ref, wlo0_ref, whi0_ref, wlo1_ref, whi1_ref,
              alo0_ref, ahi0_ref, alo1_ref, ahi1_ref,
              dlo0_ref, dhi0_ref, dlo1_ref, dhi1_ref,
              xpa_ref, adt_ref, cmax_ref, acc_ref):
    i = pl.program_id(0)

    @pl.when(i == 0)
    def _():
        acc_ref[...] = jnp.full((8, 16), -1e30, jnp.float32)

    h = h_ref[...]
    S2 = _half_selector()
    wlos = (wlo0_ref, wlo1_ref)
    whis = (whi0_ref, whi1_ref)
    alos = (alo0_ref, alo1_ref)
    ahis = (ahi0_ref, ahi1_ref)
    dlos = (dlo0_ref, dlo1_ref)
    dhis = (dhi0_ref, dhi1_ref)
    rows, adrows = [], []
    for cc in range(NC):
        xlo = jnp.dot(h, wlos[cc][...], preferred_element_type=jnp.float32)
        xhi = jnp.dot(h, whis[cc][...], preferred_element_type=jnp.float32)
        as4 = (jnp.dot(xlo * alos[cc][...], S2,
                       preferred_element_type=jnp.float32)
               + jnp.dot(xhi * ahis[cc][...], S2,
                         preferred_element_type=jnp.float32))
        ad4 = (jnp.dot(xlo * dlos[cc][...], S2,
                       preferred_element_type=jnp.float32)
               + jnp.dot(xhi * dhis[cc][...], S2,
                         preferred_element_type=jnp.float32))
        word = jnp.bitwise_or(
            lax.shift_right_logical(_bf16_bits(xlo), 16),
            jnp.bitwise_and(_bf16_bits(xhi), jnp.int32(-65536)))
        as_i = lax.bitcast_convert_type(as4, jnp.int32)
        rows.append(jnp.concatenate(
            [word, as_i, jnp.zeros((RB, XW - FH // 2 - HH), jnp.int32)],
            axis=1))
        adrows.append(jnp.concatenate(
            [jnp.zeros((RB, 8), jnp.float32), ad4,
             jnp.zeros((RB, 4), jnp.float32)], axis=1))

        acc_ref[2 * cc:2 * cc + 1, 0:HH] = jnp.maximum(
            acc_ref[2 * cc:2 * cc + 1, 0:HH],
            jnp.max(as4, axis=0, keepdims=True))
        acc_ref[2 * cc + 1:2 * cc + 2, 0:HH] = jnp.maximum(
            acc_ref[2 * cc + 1:2 * cc + 2, 0:HH],
            jnp.max(ad4, axis=0, keepdims=True))

    xpa_ref[...] = jnp.stack(rows, axis=0)
    adt_ref[...] = jnp.stack(adrows, axis=0)

    @pl.when(i == NBLK - 1)
    def _():
        big8 = jnp.full((1, 8), 100.0, jnp.float32)
        big4 = jnp.full((1, 4), 100.0, jnp.float32)
        outr = []
        for cc in range(NC):
            cm = acc_ref[2 * cc:2 * cc + 1, 0:HH] \
                + acc_ref[2 * cc + 1:2 * cc + 2, 0:HH]
            cm = jnp.maximum(cm, 0.2 * cm)
            outr.append(jnp.concatenate([big8, cm, big4], axis=1))
        cmax_ref[...] = jnp.concatenate(outr, axis=0)


def _k_pre(h, wlos, whis, alos, ahis, dlos, dhis):
    small = pl.BlockSpec((1, FH // 2), lambda i: (0, 0))
    return pl.pallas_call(
        _pre_body,
        grid=(NBLK,),
        in_specs=[
            pl.BlockSpec((RB, HID), lambda i: (i, 0)),
            pl.BlockSpec((HID, FH // 2), lambda i: (0, 0)),
            pl.BlockSpec((HID, FH // 2), lambda i: (0, 0)),
            pl.BlockSpec((HID, FH // 2), lambda i: (0, 0)),
            pl.BlockSpec((HID, FH // 2), lambda i: (0, 0)),
            small, small, small, small, small, small, small, small,
        ],
        out_specs=[
            pl.BlockSpec((NC, RB, XW), lambda i: (0, i, 0)),
            pl.BlockSpec((NC, RB, 16), lambda i: (0, i, 0)),
            pl.BlockSpec((NC, 16), lambda i: (0, 0)),
        ],
        out_shape=[
            jax.ShapeDtypeStruct((NC, N_PAD, XW), jnp.int32),
            jax.ShapeDtypeStruct((NC, N_PAD, 16), jnp.float32),
            jax.ShapeDtypeStruct((NC, 16), jnp.float32),
        ],
        scratch_shapes=[pltpu.VMEM((8, 16), jnp.float32)],
    )(h, wlos[0], whis[0], wlos[1], whis[1],
      alos[0], ahis[0], alos[1], ahis[1],
      dlos[0], dhis[0], dlos[1], dhis[1])


def _denom_expander():
    jj = lax.broadcasted_iota(jnp.int32, (HH, FH), 0)
    ff = lax.broadcasted_iota(jnp.int32, (HH, FH), 1) // DH
    return (jj == ff).astype(jnp.float32)


def _scaled_agg(agg):
    Sx = _denom_expander()
    outs = []
    for c in range(NC):
        num = agg[c, :, :FH]
        den = agg[c, :, DEN0:DEN0 + HH] + 1e-16
        rec = jnp.dot(1.0 / den, Sx, preferred_element_type=jnp.float32)
        outs.append(num * rec)
    return jnp.concatenate(outs, axis=1)


def _post_common(h_ref, agg_ref, bl_ref, lnw_ref, lnb_ref):
    hmid = h_ref[...] + _scaled_agg(agg_ref[...]) + bl_ref[...]
    m = jnp.mean(hmid, axis=1, keepdims=True)
    d = hmid - m
    v = jnp.mean(d * d, axis=1, keepdims=True)
    hn = d * lax.rsqrt(v + 1e-5) * lnw_ref[...] + lnb_ref[...]
    return jnp.maximum(hn, 0.0)


def _post_body(h_ref, agg_ref, bl_ref, lnw_ref, lnb_ref, o_ref):
    o_ref[...] = _post_common(h_ref, agg_ref, bl_ref, lnw_ref, lnb_ref)


def _k_post(h, agg, bl, lnw, lnb):
    return pl.pallas_call(
        _post_body,
        grid=(NBLK,),
        in_specs=[
            pl.BlockSpec((RB, HID), lambda i: (i, 0)),
            pl.BlockSpec((NC, RB, AW), lambda i: (0, i, 0)),
            pl.BlockSpec((1, HID), lambda i: (0, 0)),
            pl.BlockSpec((1, HID), lambda i: (0, 0)),
            pl.BlockSpec((1, HID), lambda i: (0, 0)),
        ],
        out_specs=pl.BlockSpec((RB, HID), lambda i: (i, 0)),
        out_shape=jax.ShapeDtypeStruct((N_PAD, HID), jnp.float32),
    )(h, agg, bl, lnw, lnb)


def _pool_body(h_ref, agg_ref, bl_ref, lnw_ref, lnb_ref, batch_ref, o_ref,
               sum_ref, cnt_ref):
    i = pl.program_id(0)
    hn = _post_common(h_ref, agg_ref, bl_ref, lnw_ref, lnb_ref)

    bb = batch_ref[0]
    gg = lax.broadcasted_iota(jnp.int32, (G, RB), 0)
    P = (bb == gg).astype(jnp.float32)

    @pl.when(i == 0)
    def _():
        sum_ref[...] = jnp.zeros((G, HID), jnp.float32)
        cnt_ref[...] = jnp.zeros((G, 128), jnp.float32)

    sum_ref[...] += jnp.dot(P, hn, preferred_element_type=jnp.float32)
    cnt_ref[...] += jnp.dot(P, jnp.ones((RB, 128), jnp.float32),
                            preferred_element_type=jnp.float32)

    @pl.when(i == NBLK - 1)
    def _():
        c1 = jnp.maximum(cnt_ref[...], 1.0)
        o_ref[...] = sum_ref[...] / jnp.concatenate([c1, c1], axis=1)


def _k_pool(h, agg, bl, lnw, lnb, batch3):
    return pl.pallas_call(
        _pool_body,
        grid=(NBLK,),
        in_specs=[
            pl.BlockSpec((RB, HID), lambda i: (i, 0)),
            pl.BlockSpec((NC, RB, AW), lambda i: (0, i, 0)),
            pl.BlockSpec((1, HID), lambda i: (0, 0)),
            pl.BlockSpec((1, HID), lambda i: (0, 0)),
            pl.BlockSpec((1, HID), lambda i: (0, 0)),
            pl.BlockSpec((1, 1, RB), lambda i: (i, 0, 0)),
        ],
        out_specs=pl.BlockSpec((G, HID), lambda i: (0, 0)),
        out_shape=jax.ShapeDtypeStruct((G, HID), jnp.float32),
        scratch_shapes=[pltpu.VMEM((G, HID), jnp.float32),
                        pltpu.VMEM((G, 128), jnp.float32)],
    )(h, agg, bl, lnw, lnb, batch3)


_SC_MESH = plsc.VectorSubcoreMesh(
    core_axis_name="c", subcore_axis_name="s", num_cores=NC, num_subcores=NS)

_ROWS_PER = N_PAD // NS


@functools.partial(
    pl.kernel,
    out_type=jax.ShapeDtypeStruct((NC, N_PAD, AW), jnp.float32),
    mesh=_SC_MESH,
    compiler_params=pltpu.CompilerParams(use_tc_tiling_on_sc=False),
    scratch_types=[
        pltpu.VMEM_SHARED((N_PAD, AW), jnp.float32),
        pltpu.VMEM((CB, XW), jnp.int32),
        pltpu.VMEM((CB, XW), jnp.int32),
        pltpu.VMEM((CB, AW), jnp.float32),
        pltpu.VMEM((CB, 16), jnp.float32),
        pltpu.VMEM((GRP, CB), jnp.int32),
        pltpu.VMEM((GRP, CB), jnp.int32),
        pltpu.VMEM((GRP, CB), jnp.int32),
        pltpu.VMEM((GRP, CB), jnp.int32),
        pltpu.VMEM((16,), jnp.float32),
        pltpu.SemaphoreType.DMA,
        pltpu.SemaphoreType.DMA,
        pltpu.SemaphoreType.DMA,
        pltpu.SemaphoreType.DMA,
        pltpu.SemaphoreType.DMA,
        pltpu.SemaphoreType.DMA,
    ],
)
def _sc_edge_kernel(src_hbm, dst_hbm, xpa_hbm, adt_hbm, cmax_hbm, out_hbm,
                    agg_sp, gb0, gb1, sbuf, dbuf, sg0, sg1, dg0, dg1,
                    cbuf, gs0, gs1, gd, ssem, gi0, gi1):
    c = lax.axis_index("c")
    s = lax.axis_index("s")
    gbufs = (gb0, gb1)
    gsems = (gs0, gs1)
    sgs = (sg0, sg1)
    dgs = (dg0, dg1)
    gis = (gi0, gi1)

    @pl.loop(0, CB)
    def _zero(e):
        for k in range(AW // 16):
            sbuf[e, 16 * k:16 * (k + 1)] = jnp.zeros((16,), jnp.float32)

    full, rem = divmod(_ROWS_PER, CB)
    for j in range(full):
        pltpu.sync_copy(sbuf, agg_sp.at[pl.ds(s * _ROWS_PER + j * CB, CB)])
    if rem:
        pltpu.sync_copy(sbuf.at[pl.ds(0, rem)],
                        agg_sp.at[pl.ds(s * _ROWS_PER + full * CB, rem)])
    plsc.subcore_barrier()

    pltpu.sync_copy(cmax_hbm.at[c], cbuf)
    C = cbuf[...]

    row0 = s * CHUNKS

    def issue_idx(gi, q):
        pltpu.async_copy(src_hbm.at[pl.ds(row0 + gi * GRP, GRP)], sgs[q],
                         gis[q])
        pltpu.async_copy(dst_hbm.at[pl.ds(row0 + gi * GRP, GRP)], dgs[q],
                         gis[q])

    def wait_idx(gi, q):
        pltpu.make_async_copy(src_hbm.at[pl.ds(row0 + gi * GRP, GRP)],
                              sgs[q], gis[q]).wait()
        pltpu.make_async_copy(dst_hbm.at[pl.ds(row0 + gi * GRP, GRP)],
                              dgs[q], gis[q]).wait()

    def _qr(d):
        return ((d // 2) % 2, d % 2)

    def sidx(d):
        q, r = _qr(d)
        return sgs[q].at[r]

    def didx(d):
        q, r = _qr(d)
        return dgs[q].at[r]

    def issue_gs(d, b):
        pltpu.async_copy(xpa_hbm.at[c].at[sidx(d)], gbufs[b], gsems[b])

    def wait_gs(d, b):
        pltpu.make_async_copy(xpa_hbm.at[c].at[sidx(d)], gbufs[b],
                              gsems[b]).wait()

    def issue_gd(d):
        pltpu.async_copy(adt_hbm.at[c].at[didx(d)], dbuf, gd)

    def wait_gd(d):
        pltpu.make_async_copy(adt_hbm.at[c].at[didx(d)], dbuf, gd).wait()

    def issue_sc(d):
        pltpu.async_copy(sbuf, agg_sp.at[didx(d)], ssem, add=True)

    def wait_sc(d):
        pltpu.make_async_copy(sbuf, agg_sp.at[didx(d)], ssem).wait()

    TOPMASK = jnp.int32(-65536)

    def compute_p(gbuf):
        @plsc.parallel_loop(0, CB, unroll=8)
        def _edge(e):
            arow = lax.bitcast_convert_type(gbuf[e, XW - 16:XW], jnp.float32)
            drow = dbuf[e, :]
            sm = arow + drow
            sm = jnp.maximum(sm, 0.2 * sm)
            sbuf[e, FH:FH + 16] = jnp.exp(sm - C)

    def compute_w(gbuf):
        @plsc.parallel_loop(0, CB, unroll=4)
        def _edge(e):
            p16 = sbuf[e, FH:FH + 16]
            for hh_ in range(HH):
                m = jnp.full((16,), p16[8 + hh_], jnp.float32)
                w16 = gbuf[e, 16 * hh_:16 * (hh_ + 1)]
                lo = lax.bitcast_convert_type(
                    lax.shift_left(w16, 16), jnp.float32)
                hi = lax.bitcast_convert_type(
                    jnp.bitwise_and(w16, TOPMASK), jnp.float32)
                sbuf[e, pl.ds(32 * hh_, 16)] = lo * m
                sbuf[e, pl.ds(32 * hh_ + 16, 16)] = hi * m

    issue_idx(0, 0)
    wait_idx(0, 0)
    issue_gd(0)
    issue_gs(0, 0)

    @pl.loop(0, CHUNKS // 4)
    def _super(t):
        i0 = 4 * t
        for p in range(4):
            i = i0 + p
            b = p % 2
            o = 1 - b
            if p in (1, 3):
                pl.when(i + 1 < CHUNKS)(
                    lambda i=i, p=p: wait_idx((i + 1) // GRP, _qr(p + 1)[0]))
            pl.when(i + 1 < CHUNKS)(lambda p=p, o=o: issue_gs(p + 1, o))
            wait_gs(p, b)
            wait_gd(p)
            compute_p(gbufs[b])
            pl.when(i + 1 < CHUNKS)(lambda p=p: issue_gd(p + 1))

            @pl.when(i >= 1)
            def _(p=p):
                wait_sc(p - 1)

            if p == 0:
                pl.when(i + 2 < CHUNKS)(
                    lambda i=i: issue_idx((i + 2) // GRP, 1))
            elif p == 2:
                pl.when(i + 2 < CHUNKS)(
                    lambda i=i: issue_idx((i + 2) // GRP, 0))

            compute_w(gbufs[b])
            issue_sc(p)

    wait_sc(3)

    plsc.subcore_barrier()
    for j in range(full):
        r0 = s * _ROWS_PER + j * CB
        pltpu.sync_copy(agg_sp.at[pl.ds(r0, CB)],
                        out_hbm.at[c].at[pl.ds(r0, CB)])
    if rem:
        r0 = s * _ROWS_PER + full * CB
        pltpu.sync_copy(agg_sp.at[pl.ds(r0, rem)],
                        out_hbm.at[c].at[pl.ds(r0, rem)])


def kernel(x, edge_index, batch, W_in, b_in, W_l, a_src, a_dst, b_l, ln_w,
           ln_b):
    f32 = jnp.float32
    x_pad = jnp.zeros((N_PAD, D_IN), f32).at[:N].set(x.astype(f32))
    pad_ids = jnp.full((E_PAD - E,), N, jnp.int32)
    srcp = jnp.concatenate([edge_index[0].astype(jnp.int32), pad_ids]
                           ).reshape(E_PAD // CB, CB)
    dstp = jnp.concatenate([edge_index[1].astype(jnp.int32), pad_ids]
                           ).reshape(E_PAD // CB, CB)
    batch3 = jnp.concatenate(
        [batch.astype(jnp.int32), jnp.full((N_PAD - N,), G, jnp.int32)]
    ).reshape(NBLK, 1, RB)

    b_in2 = b_in.reshape(1, HID).astype(f32)
    asv = a_src.reshape(L, HID).astype(f32)
    adv = a_dst.reshape(L, HID).astype(f32)

    def _split_w(w_half):
        r = w_half.reshape(HID, HH, 2, 16)
        return (r[:, :, 0, :].reshape(HID, FH // 2),
                r[:, :, 1, :].reshape(HID, FH // 2))

    def _split_v(v_half):
        r = v_half.reshape(HH, 2, 16)
        return (r[:, 0, :].reshape(1, FH // 2), r[:, 1, :].reshape(1, FH // 2))

    h = _k_in(x_pad, W_in.astype(f32), b_in2)
    out = None
    for l in range(L):
        wl = W_l[l].astype(f32)
        wlos, whis, alos, ahis, dlos, dhis = [], [], [], [], [], []
        for cc in range(NC):
            lo, hi = _split_w(wl[:, FH * cc:FH * (cc + 1)])
            wlos.append(lo)
            whis.append(hi)
            lo, hi = _split_v(asv[l, FH * cc:FH * (cc + 1)])
            alos.append(lo)
            ahis.append(hi)
            lo, hi = _split_v(adv[l, FH * cc:FH * (cc + 1)])
            dlos.append(lo)
            dhis.append(hi)
        xpa, adt, cmax = _k_pre(h, wlos, whis, alos, ahis, dlos, dhis)
        agg = _sc_edge_kernel(srcp, dstp, xpa, adt, cmax)
        args = (h, agg, b_l[l].reshape(1, HID).astype(f32),
                ln_w[l].reshape(1, HID).astype(f32),
                ln_b[l].reshape(1, HID).astype(f32))
        if l < L - 1:
            h = _k_post(*args)
        else:
            out = _k_pool(*args, batch3)
    return out
